# baseline (speedup 1.0000x reference)
import functools

import jax
import jax.numpy as jnp
from jax.experimental import pallas as pl
from jax.experimental.pallas import tpu as pltpu

_ALPHA = 3.0
_K = 32
_RBLK = 128
_ONE_BITS = 0x3F800000


def _nv_kernel(emb_ref, wt_ref, b_ref, out_ref):
    y = jnp.dot(emb_ref[...], wt_ref[...], preferred_element_type=jnp.float32)
    out_ref[...] = jnp.tanh(_ALPHA * (y + b_ref[...]))


def _exact_dot(x, y):
    return jax.lax.dot_general(
        x, y, (((1,), (0,)), ((), ())),
        precision=jax.lax.Precision.HIGHEST,
        preferred_element_type=jnp.float32)


def _adj_kernel(nv_ref, nvt_ref, gmap_ref, le_ref, out_ref, *, n_cols):
    a = jnp.dot(nv_ref[...], nvt_ref[...], preferred_element_type=jnp.float32)
    v = jnp.maximum(jnp.tanh(_ALPHA * a), 0.0)
    u = jax.lax.bitcast_convert_type(v, jnp.int32)

    rows, ncols_pad = v.shape
    ones_col = jnp.ones((ncols_pad, 1), jnp.float32)
    col = jax.lax.broadcasted_iota(jnp.int32, v.shape, 1)

    def body(_, carry):
        lo, hi = carry
        mid = (lo + hi + 1) >> 1
        cnt = _exact_dot((u >= mid).astype(jnp.float32), ones_col)
        ok = cnt >= float(_K)
        return jnp.where(ok, mid, lo), jnp.where(ok, hi, mid)

    def _full_search():
        lo0 = jnp.zeros((rows, 1), jnp.int32)
        hi0 = jnp.full((rows, 1), _ONE_BITS + 1, jnp.int32)
        tau_s, _ = jax.lax.fori_loop(0, 31, body, (lo0, hi0))
        return tau_s

    sat_f = (u >= _ONE_BITS).astype(jnp.float32)
    c1 = _exact_dot(sat_f, ones_col)
    all_sat = jnp.min(c1) >= float(_K)
    tau = jax.lax.cond(
        all_sat, lambda: jnp.full((rows, 1), _ONE_BITS, jnp.int32),
        _full_search)

    gt = u > tau
    cnt_gt = _exact_dot(gt.astype(jnp.float32), ones_col)
    need = float(_K) - cnt_gt
    tie = u == tau
    tie_f = tie.astype(jnp.float32)
    w = jnp.where(tie, col, ncols_pad)

    grp = _exact_dot(tie_f, gmap_ref[...])
    cum = _exact_dot(grp, le_ref[...])
    gsel = jnp.sum((cum < need).astype(jnp.int32), axis=1, keepdims=True)

    def ibody(_, carry):
        lo, hi = carry
        mid = (lo + hi) >> 1
        cnt = _exact_dot((w <= mid).astype(jnp.float32), ones_col)
        ok = cnt >= need
        return jnp.where(ok, lo, mid), jnp.where(ok, mid, hi)

    ilo0 = gsel * 128 - 1
    _, iot = jax.lax.fori_loop(0, 7, ibody, (ilo0, ilo0 + 128))

    keep = gt | (tie & (col <= iot))
    res = jnp.where(keep, v, 0.0)
    out_ref[...] = res[:, :n_cols]


def kernel(idx, emb1, W, b):
    n, d = emb1.shape
    x = jnp.take(emb1, idx, axis=0)
    npad = ((n + _RBLK - 1) // _RBLK) * _RBLK
    xp = jnp.pad(x, ((0, npad - n), (0, 0)))
    wt = W.T
    b2 = b.reshape(1, d)

    nv = pl.pallas_call(
        _nv_kernel,
        out_shape=jax.ShapeDtypeStruct((npad, d), jnp.float32),
    )(xp, wt, b2)
    nv = jnp.where(jnp.arange(npad, dtype=jnp.int32)[:, None] < n, nv, 0.0)
    nvt = nv.T

    ngrp = npad // 128
    gi = jnp.arange(ngrp, dtype=jnp.int32)
    le = (gi[:, None] <= gi[None, :]).astype(jnp.float32)
    ci = jnp.arange(npad, dtype=jnp.int32)
    gmap = ((ci[:, None] // 128) == gi[None, :]).astype(jnp.float32)

    grid = npad // _RBLK
    adj = pl.pallas_call(
        functools.partial(_adj_kernel, n_cols=n),
        grid=(grid,),
        in_specs=[
            pl.BlockSpec((_RBLK, d), lambda i: (i, 0)),
            pl.BlockSpec((d, npad), lambda i: (0, 0)),
            pl.BlockSpec((npad, ngrp), lambda i: (0, 0)),
            pl.BlockSpec((ngrp, ngrp), lambda i: (0, 0)),
        ],
        out_specs=pl.BlockSpec((_RBLK, n), lambda i: (i, 0)),
        out_shape=jax.ShapeDtypeStruct((n, n), jnp.float32),
        compiler_params=pltpu.CompilerParams(
            dimension_semantics=("parallel",)
        ),
    )(nv, nvt, gmap, le)
    return adj


# speedup vs baseline: 4.8502x; 4.8502x over previous
import functools

import jax
import jax.numpy as jnp
from jax.experimental import pallas as pl
from jax.experimental.pallas import tpu as pltpu

_ALPHA = 3.0
_K = 32
_RBLK = 128
_ONE_BITS = 0x3F800000


def _nv_kernel(emb_ref, wt_ref, b_ref, out_ref):
    y = jnp.dot(emb_ref[...], wt_ref[...], preferred_element_type=jnp.float32)
    out_ref[...] = jnp.tanh(_ALPHA * (y + b_ref[...]))


def _rowsum(x):
    return jnp.sum(x, axis=1, keepdims=True)


def _adj_kernel(nv_ref, nvt_ref, gmap_ref, le_ref, out_ref, *, n_cols):
    a = jnp.dot(nv_ref[...], nvt_ref[...], preferred_element_type=jnp.float32)
    v = jnp.maximum(jnp.tanh(_ALPHA * a), 0.0)
    u = jax.lax.bitcast_convert_type(v, jnp.int32)

    rows, ncols_pad = v.shape
    col = jax.lax.broadcasted_iota(jnp.int32, v.shape, 1)

    def body(_, carry):
        lo, hi = carry
        mid = (lo + hi + 1) >> 1
        cnt = _rowsum((u >= mid).astype(jnp.float32))
        ok = cnt >= float(_K)
        return jnp.where(ok, mid, lo), jnp.where(ok, hi, mid)

    def _full_search():
        lo0 = jnp.zeros((rows, 1), jnp.int32)
        hi0 = jnp.full((rows, 1), _ONE_BITS + 1, jnp.int32)
        tau_s, _ = jax.lax.fori_loop(0, 31, body, (lo0, hi0))
        return tau_s

    sat_f = (u >= _ONE_BITS).astype(jnp.float32)
    c1 = _rowsum(sat_f)
    all_sat = jnp.min(c1) >= float(_K)
    tau = jax.lax.cond(
        all_sat, lambda: jnp.full((rows, 1), _ONE_BITS, jnp.int32),
        _full_search)

    gt = u > tau
    cnt_gt = _rowsum(gt.astype(jnp.float32))
    need = float(_K) - cnt_gt
    tie = u == tau
    tie_f = tie.astype(jnp.float32)
    w = jnp.where(tie, col, ncols_pad)

    grp = jnp.dot(tie_f, gmap_ref[...],
                  preferred_element_type=jnp.float32)
    cum = jnp.dot(grp, le_ref[...],
                  preferred_element_type=jnp.float32)
    gsel = jnp.sum((cum < need).astype(jnp.int32), axis=1, keepdims=True)

    def ibody(_, carry):
        lo, hi = carry
        mid = (lo + hi) >> 1
        cnt = _rowsum((w <= mid).astype(jnp.float32))
        ok = cnt >= need
        return jnp.where(ok, lo, mid), jnp.where(ok, mid, hi)

    ilo0 = gsel * 128 - 1
    _, iot = jax.lax.fori_loop(0, 7, ibody, (ilo0, ilo0 + 128))

    keep = gt | (tie & (col <= iot))
    res = jnp.where(keep, v, 0.0)
    out_ref[...] = res[:, :n_cols]


def kernel(idx, emb1, W, b):
    n, d = emb1.shape
    x = jnp.take(emb1, idx, axis=0)
    npad = ((n + _RBLK - 1) // _RBLK) * _RBLK
    xp = jnp.pad(x, ((0, npad - n), (0, 0)))
    wt = W.T
    b2 = b.reshape(1, d)

    nv = pl.pallas_call(
        _nv_kernel,
        out_shape=jax.ShapeDtypeStruct((npad, d), jnp.float32),
    )(xp, wt, b2)
    nv = jnp.where(jnp.arange(npad, dtype=jnp.int32)[:, None] < n, nv, 0.0)
    nvt = nv.T

    ngrp = npad // 128
    gi = jnp.arange(ngrp, dtype=jnp.int32)
    le = (gi[:, None] <= gi[None, :]).astype(jnp.float32)
    ci = jnp.arange(npad, dtype=jnp.int32)
    gmap = ((ci[:, None] // 128) == gi[None, :]).astype(jnp.float32)

    grid = npad // _RBLK
    adj = pl.pallas_call(
        functools.partial(_adj_kernel, n_cols=n),
        grid=(grid,),
        in_specs=[
            pl.BlockSpec((_RBLK, d), lambda i: (i, 0)),
            pl.BlockSpec((d, npad), lambda i: (0, 0)),
            pl.BlockSpec((npad, ngrp), lambda i: (0, 0)),
            pl.BlockSpec((ngrp, ngrp), lambda i: (0, 0)),
        ],
        out_specs=pl.BlockSpec((_RBLK, n), lambda i: (i, 0)),
        out_shape=jax.ShapeDtypeStruct((n, n), jnp.float32),
        compiler_params=pltpu.CompilerParams(
            dimension_semantics=("parallel",)
        ),
    )(nv, nvt, gmap, le)
    return adj


# speedup vs baseline: 8.1217x; 1.6745x over previous
import functools

import jax
import jax.numpy as jnp
from jax.experimental import pallas as pl
from jax.experimental.pallas import tpu as pltpu

_ALPHA = 3.0
_K = 32
_RBLK = 128
_ONE_BITS = 0x3F800000


def _nv_kernel(emb_ref, wt_ref, b_ref, out_ref):
    y = jnp.dot(emb_ref[...], wt_ref[...], preferred_element_type=jnp.float32)
    out_ref[...] = jnp.tanh(_ALPHA * (y + b_ref[...]))


def _rowsum(x):
    return jnp.sum(x, axis=1, keepdims=True)


def _adj_kernel(nv_ref, nvt_ref, gmap_ref, le_ref, pr_ref, out_ref, *,
                n_cols):
    a = jnp.dot(nv_ref[...], nvt_ref[...], preferred_element_type=jnp.float32)
    v = jnp.maximum(jnp.tanh(_ALPHA * a), 0.0)
    u = jax.lax.bitcast_convert_type(v, jnp.int32)

    rows, ncols_pad = v.shape
    ngrp = ncols_pad // 128
    col = jax.lax.broadcasted_iota(jnp.int32, v.shape, 1)

    def body(_, carry):
        lo, hi = carry
        mid = (lo + hi + 1) >> 1
        cnt = _rowsum((u >= mid).astype(jnp.float32))
        ok = cnt >= float(_K)
        return jnp.where(ok, mid, lo), jnp.where(ok, hi, mid)

    def _full_search():
        lo0 = jnp.zeros((rows, 1), jnp.int32)
        hi0 = jnp.full((rows, 1), _ONE_BITS + 1, jnp.int32)
        tau_s, _ = jax.lax.fori_loop(0, 31, body, (lo0, hi0))
        return tau_s

    sat_f = (u >= _ONE_BITS).astype(jnp.float32)
    c1 = _rowsum(sat_f)
    all_sat = jnp.min(c1) >= float(_K)
    tau = jax.lax.cond(
        all_sat, lambda: jnp.full((rows, 1), _ONE_BITS, jnp.int32),
        _full_search)

    gt = u > tau
    cnt_gt = _rowsum(gt.astype(jnp.float32))
    need = float(_K) - cnt_gt
    tie = u == tau
    tie_f = tie.astype(jnp.float32)
    w = jnp.where(tie, col, ncols_pad)

    grp = jnp.dot(tie_f, gmap_ref[...],
                  preferred_element_type=jnp.float32)
    cum = jnp.dot(grp, le_ref[...],
                  preferred_element_type=jnp.float32)
    gsel = jnp.sum((cum < need).astype(jnp.int32), axis=1, keepdims=True)

    gid = jax.lax.broadcasted_iota(jnp.int32, (rows, ngrp), 1)
    cumprev = _rowsum(jnp.where(gid < gsel, grp, 0.0))
    need2 = need - cumprev
    tsel = jnp.where((col >> 7) == gsel, tie_f, 0.0)
    p = jnp.dot(tsel, pr_ref[...],
                preferred_element_type=jnp.float32)
    lane = _rowsum((p < need2).astype(jnp.float32)).astype(jnp.int32)
    iot = gsel * 128 + lane

    keep = gt | (w <= iot)
    res = jnp.where(keep, v, 0.0)
    out_ref[...] = res[:, :n_cols]


def kernel(idx, emb1, W, b):
    n, d = emb1.shape
    x = jnp.take(emb1, idx, axis=0)
    npad = ((n + _RBLK - 1) // _RBLK) * _RBLK
    xp = jnp.pad(x, ((0, npad - n), (0, 0)))
    wt = W.T
    b2 = b.reshape(1, d)

    nv = pl.pallas_call(
        _nv_kernel,
        out_shape=jax.ShapeDtypeStruct((npad, d), jnp.float32),
    )(xp, wt, b2)
    nv = jnp.where(jnp.arange(npad, dtype=jnp.int32)[:, None] < n, nv, 0.0)
    nvt = nv.T

    ngrp = npad // 128
    gi = jnp.arange(ngrp, dtype=jnp.int32)
    le = (gi[:, None] <= gi[None, :]).astype(jnp.float32)
    ci = jnp.arange(npad, dtype=jnp.int32)
    gmap = ((ci[:, None] // 128) == gi[None, :]).astype(jnp.float32)
    pr = ((ci[:, None] % 128) <= jnp.arange(128, dtype=jnp.int32)[None, :]
          ).astype(jnp.float32)

    grid = npad // _RBLK
    adj = pl.pallas_call(
        functools.partial(_adj_kernel, n_cols=n),
        grid=(grid,),
        in_specs=[
            pl.BlockSpec((_RBLK, d), lambda i: (i, 0)),
            pl.BlockSpec((d, npad), lambda i: (0, 0)),
            pl.BlockSpec((npad, ngrp), lambda i: (0, 0)),
            pl.BlockSpec((ngrp, ngrp), lambda i: (0, 0)),
            pl.BlockSpec((npad, 128), lambda i: (0, 0)),
        ],
        out_specs=pl.BlockSpec((_RBLK, n), lambda i: (i, 0)),
        out_shape=jax.ShapeDtypeStruct((n, n), jnp.float32),
        compiler_params=pltpu.CompilerParams(
            dimension_semantics=("parallel",)
        ),
    )(nv, nvt, gmap, le, pr)
    return adj
